# baseline (speedup 1.0000x reference)
import functools

import jax
import jax.numpy as jnp
from jax import lax
from jax.experimental import pallas as pl
from jax.experimental.pallas import tpu as pltpu
from jax.experimental.pallas import tpu_sc as plsc

N = 10000
E = 320000
D = 128
NC = 2
NS = 16
NW = NC * NS
EPW = E // NW
CH = 80
NCHUNK = EPW // CH
SB = 5
NSB = NCHUNK // SB
NPAD = 10240
RPT = NPAD // NS


def _sc_scatter_body(x_hbm, src_hbm, dst2_hbm, ea_hbm, out_hbm, *scr):
    sidx = scr[0:2]
    didx = scr[2:4]
    ea = scr[4:6]
    xg = scr[6:8]
    acc_sh = scr[8]
    semi = scr[9:11]
    seme = scr[11:13]
    semg = scr[13:15]

    cid = lax.axis_index("c")
    sid = lax.axis_index("s")
    wid = cid * NS + sid
    base = wid * EPW

    zero16 = jnp.zeros((16,), jnp.float32)

    def zrow(r, c):
        for j in range(8):
            ea[0][r, pl.ds(j * 16, 16)] = zero16
        return c

    lax.fori_loop(0, CH, zrow, 0)
    for kb in range(RPT // CH):
        pltpu.sync_copy(ea[0], acc_sh.at[pl.ds(sid * RPT + kb * CH, CH)])
    plsc.subcore_barrier()

    def start_i(s, b):
        pltpu.async_copy(src_hbm.at[pl.ds(base + s * SB * CH, SB * CH)],
                         sidx[b], semi[b])
        pltpu.async_copy(dst2_hbm.at[wid * NSB + s], didx[b], semi[b])

    def wait_i(b):
        pltpu.make_async_copy(src_hbm.at[pl.ds(0, SB * CH)], sidx[b],
                              semi[b]).wait()
        pltpu.make_async_copy(dst2_hbm.at[0], didx[b], semi[b]).wait()

    def start_a(g, b):
        pltpu.async_copy(ea_hbm.at[pl.ds(base + g * CH, CH)], ea[b], seme[b])

    def wait_a(b):
        pltpu.make_async_copy(ea_hbm.at[pl.ds(0, CH)], ea[b], seme[b]).wait()

    def start_g(b, ib, jj):
        pltpu.async_copy(x_hbm.at[sidx[ib].at[pl.ds(jj * CH, CH)]], xg[b],
                         semg[b])

    def wait_g(b, ib, jj):
        pltpu.make_async_copy(x_hbm.at[sidx[ib].at[pl.ds(jj * CH, CH)]],
                              xg[b], semg[b]).wait()

    def compute(b):
        def row(r, cc):
            for j in range(8):
                s = pl.ds(j * 16, 16)
                ea[b][r, s] = jnp.maximum(ea[b][r, s] + xg[b][r, s], 0.0)
            return cc

        lax.fori_loop(0, CH, row, 0)

    start_i(0, 0)
    start_a(0, 0)
    wait_i(0)
    wait_a(0)
    start_g(0, 0, 0)

    def outer(k, c):
        g0 = 10 * k
        for j in range(10):
            g = g0 + j
            b = j % 2
            ib = (j // SB) % 2
            jj = j % SB
            nib = ((j + 1) // SB) % 2
            njj = (j + 1) % SB
            if j == 0:
                start_i(2 * k + 1, 1)
            if j == SB:
                start_i(2 * k + 2, 0)
            if j == SB - 1:
                wait_i(1)
            if j == 2 * SB - 1:
                wait_i(0)
            start_a(g + 1, 1 - b)
            wait_g(b, ib, jj)
            compute(b)
            wait_a(1 - b)
            start_g(1 - b, nib, njj)
            pltpu.sync_copy(ea[b], acc_sh.at[didx[ib].at[jj]], add=True)
        return c

    lax.fori_loop(0, (NSB - 1) // 2, outer, 0)

    for j in range(SB):
        g = (NSB - 1) * SB + j
        b = j % 2
        if j < SB - 1:
            start_a(g + 1, 1 - b)
        wait_g(b, 0, j)
        compute(b)
        if j < SB - 1:
            wait_a(1 - b)
            start_g(1 - b, 0, j + 1)
        pltpu.sync_copy(ea[b], acc_sh.at[didx[0].at[j]], add=True)

    plsc.subcore_barrier()
    pltpu.sync_copy(acc_sh.at[pl.ds(sid * RPT, RPT)],
                    out_hbm.at[cid, pl.ds(sid * RPT, RPT)])


_sc_scatter = functools.partial(
    pl.kernel,
    out_type=jax.ShapeDtypeStruct((NC, NPAD, D), jnp.float32),
    mesh=plsc.VectorSubcoreMesh(core_axis_name="c", subcore_axis_name="s"),
    scratch_types=(
        [pltpu.VMEM((SB * CH,), jnp.int32)] * 2
        + [pltpu.VMEM((SB, CH), jnp.int32)] * 2
        + [pltpu.VMEM((CH, D), jnp.float32)] * 2
        + [pltpu.VMEM((CH, D), jnp.float32)] * 2
        + [pltpu.VMEM_SHARED((NPAD, D), jnp.float32)]
        + [pltpu.SemaphoreType.DMA] * 6
    ),
)(_sc_scatter_body)


def _sc_reduce_body(x_hbm, src_hbm, ea_hbm, out_hbm, *scr):
    sidx = scr[0:2]
    ea = scr[2:4]
    xg = scr[4:6]
    obuf = scr[6]
    semi = scr[7:9]
    seme = scr[9:11]
    semg = scr[11:13]

    cid = lax.axis_index("c")
    sid = lax.axis_index("s")
    wid = cid * NS + sid
    base = wid * EPW

    def start_i(s, b):
        pltpu.async_copy(src_hbm.at[pl.ds(base + s * SB * CH, SB * CH)],
                         sidx[b], semi[b])

    def wait_i(b):
        pltpu.make_async_copy(src_hbm.at[pl.ds(0, SB * CH)], sidx[b],
                              semi[b]).wait()

    def start_a(g, b):
        pltpu.async_copy(ea_hbm.at[pl.ds(base + g * CH, CH)], ea[b], seme[b])

    def wait_a(b):
        pltpu.make_async_copy(ea_hbm.at[pl.ds(0, CH)], ea[b], seme[b]).wait()

    def start_g(b, ib, jj):
        pltpu.async_copy(x_hbm.at[sidx[ib].at[pl.ds(jj * CH, CH)]], xg[b],
                         semg[b])

    def wait_g(b, ib, jj):
        pltpu.make_async_copy(x_hbm.at[sidx[ib].at[pl.ds(jj * CH, CH)]],
                              xg[b], semg[b]).wait()

    def compute(b, acc):
        def row(r, a):
            out = []
            for j in range(8):
                s = pl.ds(j * 16, 16)
                out.append(a[j] + jnp.maximum(ea[b][r, s] + xg[b][r, s], 0.0))
            return tuple(out)

        return lax.fori_loop(0, CH, row, acc)

    start_i(0, 0)
    start_a(0, 0)
    wait_i(0)
    wait_a(0)
    start_g(0, 0, 0)

    acc0 = tuple(jnp.zeros((16,), jnp.float32) for _ in range(8))

    def outer(k, acc):
        for j in range(10):
            g = 10 * k + j
            b = j % 2
            ib = (j // SB) % 2
            jj = j % SB
            nib = ((j + 1) // SB) % 2
            njj = (j + 1) % SB
            if j == 0:
                start_i(2 * k + 1, 1)
            if j == SB:
                start_i(2 * k + 2, 0)
            if j == SB - 1:
                wait_i(1)
            if j == 2 * SB - 1:
                wait_i(0)
            start_a(g + 1, 1 - b)
            wait_g(b, ib, jj)
            wait_a(1 - b)
            start_g(1 - b, nib, njj)
            acc = compute(b, acc)
        return acc

    acc = lax.fori_loop(0, (NSB - 1) // 2, outer, acc0)

    for j in range(SB):
        b = j % 2
        g = (NSB - 1) * SB + j
        if j < SB - 1:
            start_a(g + 1, 1 - b)
        wait_g(b, 0, j)
        if j < SB - 1:
            wait_a(1 - b)
            start_g(1 - b, 0, j + 1)
        acc = compute(b, acc)

    for j in range(8):
        obuf[pl.ds(j * 16, 16)] = acc[j]
    pltpu.sync_copy(obuf, out_hbm.at[pl.ds(wid * D, D)])


_sc_reduce = functools.partial(
    pl.kernel,
    out_type=jax.ShapeDtypeStruct((NW * D,), jnp.float32),
    mesh=plsc.VectorSubcoreMesh(core_axis_name="c", subcore_axis_name="s"),
    scratch_types=(
        [pltpu.VMEM((SB * CH,), jnp.int32)] * 2
        + [pltpu.VMEM((CH, D), jnp.float32)] * 2
        + [pltpu.VMEM((CH, D), jnp.float32)] * 2
        + [pltpu.VMEM((D,), jnp.float32)]
        + [pltpu.SemaphoreType.DMA] * 6
    ),
)(_sc_reduce_body)


BLK = 400


def _tc_layer_body(x_ref, a0_ref, a1_ref, w_ref, b_ref, out_ref, cs_ref):
    i = pl.program_id(0)
    h = x_ref[...] + a0_ref[...] + a1_ref[...]
    o = jnp.dot(h, w_ref[...], preferred_element_type=jnp.float32) + b_ref[...]
    out_ref[...] = o

    @pl.when(i == 0)
    def _():
        cs_ref[...] = jnp.zeros_like(cs_ref)

    cs_ref[...] += jnp.sum(o, axis=0, keepdims=True)


def _tc_layer(x, a0, a1, w, b):
    return pl.pallas_call(
        _tc_layer_body,
        grid=(N // BLK,),
        in_specs=[
            pl.BlockSpec((BLK, D), lambda i: (i, 0)),
            pl.BlockSpec((BLK, D), lambda i: (i, 0)),
            pl.BlockSpec((BLK, D), lambda i: (i, 0)),
            pl.BlockSpec((D, D), lambda i: (0, 0)),
            pl.BlockSpec((1, D), lambda i: (0, 0)),
        ],
        out_specs=[
            pl.BlockSpec((BLK, D), lambda i: (i, 0)),
            pl.BlockSpec((1, D), lambda i: (0, 0)),
        ],
        out_shape=[
            jax.ShapeDtypeStruct((N, D), jnp.float32),
            jax.ShapeDtypeStruct((1, D), jnp.float32),
        ],
    )(x, a0, a1, w, b)


def _tc_final_body(cs_ref, s3_ref, w3_ref, b3_ref, wo_ref, bo_ref, out_ref):
    colsum_h3 = cs_ref[...] + jnp.sum(s3_ref[...], axis=0, keepdims=True)
    pooled = (colsum_h3 * (1.0 / N)) @ w3_ref[...] + b3_ref[...]
    out_ref[...] = jnp.tanh(
        jnp.dot(pooled, wo_ref[...], preferred_element_type=jnp.float32)
        + bo_ref[...])


def _tc_final(cs, s3, w3, b3, w_out, b_out):
    return pl.pallas_call(
        _tc_final_body,
        out_shape=jax.ShapeDtypeStruct((1, 256), jnp.float32),
    )(cs, s3, w3, b3, w_out, b_out)


def kernel(x, edge_index, edge_attr, W1, b1, W2, b2, W3, b3, W_out, b_out):
    src = edge_index[0].astype(jnp.int32)
    dst = edge_index[1].astype(jnp.int32)
    dst2 = dst.reshape(NW * NSB, SB, CH)
    b1r = b1.reshape(1, D)
    b2r = b2.reshape(1, D)
    b3r = b3.reshape(1, D)
    bor = b_out.reshape(1, 256)

    agg = _sc_scatter(x, src, dst2, edge_attr)
    h, _ = _tc_layer(x, agg[0, :N], agg[1, :N], W1, b1r)
    agg = _sc_scatter(h, src, dst2, edge_attr)
    h, cs2 = _tc_layer(h, agg[0, :N], agg[1, :N], W2, b2r)
    s3 = _sc_reduce(h, src, edge_attr).reshape(NW, D)
    return _tc_final(cs2, s3, W3, b3r, W_out, bor)


# speedup vs baseline: 1.3483x; 1.3483x over previous
import functools

import jax
import jax.numpy as jnp
from jax import lax
from jax.experimental import pallas as pl
from jax.experimental.pallas import tpu as pltpu
from jax.experimental.pallas import tpu_sc as plsc

N = 10000
E = 320000
D = 128
NC = 2
NS = 16
NW = NC * NS
EPW = E // NW
CH = 80
NCHUNK = EPW // CH
SB = 5
NSB = NCHUNK // SB
NPAD = 10240
RPT = NPAD // NS


def _sc_scatter_body(x_hbm, src_hbm, dst2_hbm, ea_hbm, out_hbm, *scr):
    sidx = scr[0:2]
    didx = scr[2:4]
    ea = scr[4:6]
    xg = scr[6:8]
    acc_sh = scr[8]
    semi = scr[9:11]
    seme = scr[11:13]
    semg = scr[13:15]
    semd = scr[15:17]

    cid = lax.axis_index("c")
    sid = lax.axis_index("s")
    wid = cid * NS + sid
    base = wid * EPW

    zero16 = jnp.zeros((16,), jnp.float32)

    def zrow(r, c):
        for j in range(8):
            ea[0][r, pl.ds(j * 16, 16)] = zero16
        return c

    lax.fori_loop(0, CH, zrow, 0)
    for kb in range(RPT // CH):
        pltpu.sync_copy(ea[0], acc_sh.at[pl.ds(sid * RPT + kb * CH, CH)])
    plsc.subcore_barrier()

    def start_i(s, b):
        pltpu.async_copy(src_hbm.at[pl.ds(base + s * SB * CH, SB * CH)],
                         sidx[b], semi[b])
        pltpu.async_copy(dst2_hbm.at[wid * NSB + s], didx[b], semi[b])

    def wait_i(b):
        pltpu.make_async_copy(src_hbm.at[pl.ds(0, SB * CH)], sidx[b],
                              semi[b]).wait()
        pltpu.make_async_copy(dst2_hbm.at[0], didx[b], semi[b]).wait()

    def start_a(g, b):
        pltpu.async_copy(ea_hbm.at[pl.ds(base + g * CH, CH)], ea[b], seme[b])

    def wait_a(b):
        pltpu.make_async_copy(ea_hbm.at[pl.ds(0, CH)], ea[b], seme[b]).wait()

    def start_g(b, ib, jj):
        pltpu.async_copy(x_hbm.at[sidx[ib].at[pl.ds(jj * CH, CH)]], xg[b],
                         semg[b])

    def wait_g(b, ib, jj):
        pltpu.make_async_copy(x_hbm.at[sidx[ib].at[pl.ds(jj * CH, CH)]],
                              xg[b], semg[b]).wait()

    def start_d(b, ib, jj):
        pltpu.async_copy(ea[b], acc_sh.at[didx[ib].at[jj]], semd[b], add=True)

    def wait_d(b, ib, jj):
        pltpu.make_async_copy(ea[b], acc_sh.at[didx[ib].at[jj]],
                              semd[b]).wait()

    def compute(b):
        def row(r, cc):
            for j in range(8):
                s = pl.ds(j * 16, 16)
                ea[b][r, s] = jnp.maximum(ea[b][r, s] + xg[b][r, s], 0.0)
            return cc

        lax.fori_loop(0, CH, row, 0)

    start_i(0, 0)
    start_a(0, 0)
    wait_i(0)
    start_g(0, 0, 0)

    def outer(k, c):
        g0 = 10 * k
        for j in range(10):
            g = g0 + j
            b = j % 2
            ib = (j // SB) % 2
            jj = j % SB
            nib = ((j + 1) // SB) % 2
            njj = (j + 1) % SB
            if j == 0:
                @pl.when(k > 0)
                def _():
                    wait_d(1, 1, SB - 1)
                start_i(2 * k + 1, 1)
            else:
                wait_d(1 - b, ib if jj > 0 else 1 - ib, (jj - 1) % SB)
            if j == SB:
                start_i(2 * k + 2, 0)
            if j == SB - 2:
                wait_i(1)
            if j == 2 * SB - 2:
                wait_i(0)
            start_a(g + 1, 1 - b)
            start_g(1 - b, nib, njj)
            wait_a(b)
            wait_g(b, ib, jj)
            compute(b)
            start_d(b, ib, jj)
        return c

    lax.fori_loop(0, (NSB - 1) // 2, outer, 0)

    for j in range(SB):
        g = (NSB - 1) * SB + j
        b = j % 2
        if j == 0:
            wait_d(1, 1, SB - 1)
        else:
            wait_d(1 - b, 0, j - 1)
        if j < SB - 1:
            start_a(g + 1, 1 - b)
            start_g(1 - b, 0, j + 1)
        wait_a(b)
        wait_g(b, 0, j)
        compute(b)
        start_d(b, 0, j)
    wait_d((SB - 1) % 2, 0, SB - 1)

    plsc.subcore_barrier()
    pltpu.sync_copy(acc_sh.at[pl.ds(sid * RPT, RPT)],
                    out_hbm.at[cid, pl.ds(sid * RPT, RPT)])


_sc_scatter = functools.partial(
    pl.kernel,
    out_type=jax.ShapeDtypeStruct((NC, NPAD, D), jnp.float32),
    mesh=plsc.VectorSubcoreMesh(core_axis_name="c", subcore_axis_name="s"),
    scratch_types=(
        [pltpu.VMEM((SB * CH,), jnp.int32)] * 2
        + [pltpu.VMEM((SB, CH), jnp.int32)] * 2
        + [pltpu.VMEM((CH, D), jnp.float32)] * 2
        + [pltpu.VMEM((CH, D), jnp.float32)] * 2
        + [pltpu.VMEM_SHARED((NPAD, D), jnp.float32)]
        + [pltpu.SemaphoreType.DMA] * 8
    ),
)(_sc_scatter_body)


def _sc_reduce_body(x_hbm, src_hbm, ea_hbm, out_hbm, *scr):
    sidx = scr[0:2]
    ea = scr[2:4]
    xg = scr[4:6]
    obuf = scr[6]
    semi = scr[7:9]
    seme = scr[9:11]
    semg = scr[11:13]

    cid = lax.axis_index("c")
    sid = lax.axis_index("s")
    wid = cid * NS + sid
    base = wid * EPW

    def start_i(s, b):
        pltpu.async_copy(src_hbm.at[pl.ds(base + s * SB * CH, SB * CH)],
                         sidx[b], semi[b])

    def wait_i(b):
        pltpu.make_async_copy(src_hbm.at[pl.ds(0, SB * CH)], sidx[b],
                              semi[b]).wait()

    def start_a(g, b):
        pltpu.async_copy(ea_hbm.at[pl.ds(base + g * CH, CH)], ea[b], seme[b])

    def wait_a(b):
        pltpu.make_async_copy(ea_hbm.at[pl.ds(0, CH)], ea[b], seme[b]).wait()

    def start_g(b, ib, jj):
        pltpu.async_copy(x_hbm.at[sidx[ib].at[pl.ds(jj * CH, CH)]], xg[b],
                         semg[b])

    def wait_g(b, ib, jj):
        pltpu.make_async_copy(x_hbm.at[sidx[ib].at[pl.ds(jj * CH, CH)]],
                              xg[b], semg[b]).wait()

    def compute(b, acc):
        def row(r, a):
            out = []
            for j in range(8):
                s = pl.ds(j * 16, 16)
                out.append(a[j] + jnp.maximum(ea[b][r, s] + xg[b][r, s], 0.0))
            return tuple(out)

        return lax.fori_loop(0, CH, row, acc)

    start_i(0, 0)
    start_a(0, 0)
    wait_i(0)
    start_g(0, 0, 0)

    acc0 = tuple(jnp.zeros((16,), jnp.float32) for _ in range(8))

    def outer(k, acc):
        for j in range(10):
            g = 10 * k + j
            b = j % 2
            ib = (j // SB) % 2
            jj = j % SB
            nib = ((j + 1) // SB) % 2
            njj = (j + 1) % SB
            if j == 0:
                start_i(2 * k + 1, 1)
            if j == SB:
                start_i(2 * k + 2, 0)
            if j == SB - 2:
                wait_i(1)
            if j == 2 * SB - 2:
                wait_i(0)
            start_a(g + 1, 1 - b)
            start_g(1 - b, nib, njj)
            wait_a(b)
            wait_g(b, ib, jj)
            acc = compute(b, acc)
        return acc

    acc = lax.fori_loop(0, (NSB - 1) // 2, outer, acc0)

    for j in range(SB):
        b = j % 2
        g = (NSB - 1) * SB + j
        if j < SB - 1:
            start_a(g + 1, 1 - b)
            start_g(1 - b, 0, j + 1)
        wait_a(b)
        wait_g(b, 0, j)
        acc = compute(b, acc)

    for j in range(8):
        obuf[pl.ds(j * 16, 16)] = acc[j]
    pltpu.sync_copy(obuf, out_hbm.at[pl.ds(wid * D, D)])


_sc_reduce = functools.partial(
    pl.kernel,
    out_type=jax.ShapeDtypeStruct((NW * D,), jnp.float32),
    mesh=plsc.VectorSubcoreMesh(core_axis_name="c", subcore_axis_name="s"),
    scratch_types=(
        [pltpu.VMEM((SB * CH,), jnp.int32)] * 2
        + [pltpu.VMEM((CH, D), jnp.float32)] * 2
        + [pltpu.VMEM((CH, D), jnp.float32)] * 2
        + [pltpu.VMEM((D,), jnp.float32)]
        + [pltpu.SemaphoreType.DMA] * 6
    ),
)(_sc_reduce_body)


BLK = 400


def _tc_layer_body(x_ref, a_ref, w_ref, b_ref, out_ref, cs_ref):
    i = pl.program_id(0)
    h = x_ref[...] + a_ref[0] + a_ref[1]
    o = jnp.dot(h, w_ref[...], preferred_element_type=jnp.float32) + b_ref[...]
    out_ref[...] = o

    @pl.when(i == 0)
    def _():
        cs_ref[...] = jnp.zeros_like(cs_ref)

    cs_ref[...] += jnp.sum(o, axis=0, keepdims=True)


def _tc_layer(x, agg, w, b):
    return pl.pallas_call(
        _tc_layer_body,
        grid=(N // BLK,),
        in_specs=[
            pl.BlockSpec((BLK, D), lambda i: (i, 0)),
            pl.BlockSpec((NC, BLK, D), lambda i: (0, i, 0)),
            pl.BlockSpec((D, D), lambda i: (0, 0)),
            pl.BlockSpec((1, D), lambda i: (0, 0)),
        ],
        out_specs=[
            pl.BlockSpec((BLK, D), lambda i: (i, 0)),
            pl.BlockSpec((1, D), lambda i: (0, 0)),
        ],
        out_shape=[
            jax.ShapeDtypeStruct((N, D), jnp.float32),
            jax.ShapeDtypeStruct((1, D), jnp.float32),
        ],
    )(x, agg, w, b)


def _tc_final_body(cs_ref, s3_ref, w3_ref, b3_ref, wo_ref, bo_ref, out_ref):
    colsum_h3 = cs_ref[...] + jnp.sum(s3_ref[...], axis=0, keepdims=True)
    pooled = (colsum_h3 * (1.0 / N)) @ w3_ref[...] + b3_ref[...]
    out_ref[...] = jnp.tanh(
        jnp.dot(pooled, wo_ref[...], preferred_element_type=jnp.float32)
        + bo_ref[...])


def _tc_final(cs, s3, w3, b3, w_out, b_out):
    return pl.pallas_call(
        _tc_final_body,
        out_shape=jax.ShapeDtypeStruct((1, 256), jnp.float32),
    )(cs, s3, w3, b3, w_out, b_out)


def kernel(x, edge_index, edge_attr, W1, b1, W2, b2, W3, b3, W_out, b_out):
    src = edge_index[0].astype(jnp.int32)
    dst = edge_index[1].astype(jnp.int32)
    dst2 = dst.reshape(NW * NSB, SB, CH)
    b1r = b1.reshape(1, D)
    b2r = b2.reshape(1, D)
    b3r = b3.reshape(1, D)
    bor = b_out.reshape(1, 256)

    agg = _sc_scatter(x, src, dst2, edge_attr)
    h, _ = _tc_layer(x, agg, W1, b1r)
    agg = _sc_scatter(h, src, dst2, edge_attr)
    h, cs2 = _tc_layer(h, agg, W2, b2r)
    s3 = _sc_reduce(h, src, edge_attr).reshape(NW, D)
    return _tc_final(cs2, s3, W3, b3r, W_out, bor)
